# baseline (speedup 1.0000x reference)
import functools

import jax
import jax.numpy as jnp
from jax import lax
from jax.experimental import pallas as pl
from jax.experimental.pallas import tpu as pltpu
from jax.experimental.pallas import tpu_sc as plsc

EMB = 512
NSEQ = 16
MAXLEN = 3072
TOTAL = 32768
LEN_EVEN = 1024
PAIR = 4096

NW = 32
SROWS = 4
NSLAB_LO = LEN_EVEN // NW // SROWS
NSLAB = NSLAB_LO + (MAXLEN - LEN_EVEN) // NW // SROWS
NBUF = 3
LANE = 16
VPR = EMB // LANE

_X_OFF = [(b >> 1) * PAIR + (b & 1) * LEN_EVEN for b in range(NSEQ)]


def _pe_pad_body(x_hbm, pe_hbm, out_hbm,
                 ob0, ob1, ob2, pb0, pb1, pb2,
                 si0, si1, si2, so0, so1, so2):
    wid = lax.axis_index("s") * 2 + lax.axis_index("c")

    obs = (ob0, ob1, ob2)
    pbs = (pb0, pb1, pb2)
    in_sems = (si0, si1, si2)
    out_sems = (so0, so1, so2)

    zero = jnp.zeros((LANE,), jnp.float32)
    lo0 = wid * (LEN_EVEN // NW)
    hi0 = LEN_EVEN + wid * ((MAXLEN - LEN_EVEN) // NW)

    def slab_ss(k):
        return jnp.where(k < NSLAB_LO,
                         lo0 + k * SROWS,
                         hi0 + (k - NSLAB_LO) * SROWS)

    def out_copy(par, ss):
        return pltpu.make_async_copy(
            obs[par], out_hbm.at[pl.ds(ss, SROWS)], out_sems[par])

    def fire_reads(par, k):
        ss = slab_ss(k)
        pltpu.make_async_copy(
            pe_hbm.at[pl.ds(ss, SROWS)], pbs[par], in_sems[par]).start()
        for b in range(1, NSEQ, 2):
            pltpu.make_async_copy(
                x_hbm.at[pl.ds(_X_OFF[b] + ss, SROWS)],
                obs[par].at[:, pl.ds(b, 1)], in_sems[par]).start()

        @pl.when(k < NSLAB_LO)
        def _():
            for b in range(0, NSEQ, 2):
                pltpu.make_async_copy(
                    x_hbm.at[pl.ds(_X_OFF[b] + ss, SROWS)],
                    obs[par].at[:, pl.ds(b, 1)], in_sems[par]).start()

    fire_reads(0, 0)

    def group(g, _):
        for r in range(NBUF):
            k = NBUF * g + r
            ss = slab_ss(k)
            ev = k < NSLAB_LO

            r1 = (r + 1) % NBUF

            @pl.when(k + 1 < NSLAB)
            def _():
                @pl.when(k + 1 >= NBUF)
                def _():
                    out_copy(r1, 0).wait()

                fire_reads(r1, k + 1)

            @pl.when((k >= NSLAB_LO) & (k < NSLAB_LO + NBUF))
            def _zero_even():
                def zj(j, _):
                    sl = pl.ds(j * LANE, LANE)
                    for s in range(SROWS):
                        for b in range(0, NSEQ, 2):
                            obs[r][s, b, sl] = zero
                    return 0

                lax.fori_loop(0, VPR, zj, 0)

            pltpu.make_async_copy(
                pe_hbm.at[pl.ds(0, SROWS)], pbs[r], in_sems[r]).wait()

            @pl.when(ev)
            def _():
                pltpu.make_async_copy(
                    out_hbm.at[pl.ds(0, SROWS)], obs[r], in_sems[r]).wait()

            @pl.when(jnp.logical_not(ev))
            def _():
                pltpu.make_async_copy(
                    out_hbm.at[pl.ds(0, SROWS), pl.ds(0, NSEQ // 2)],
                    obs[r].at[:, pl.ds(0, NSEQ // 2)], in_sems[r]).wait()

            for sh in range(2):
                rows = range(2 * sh, 2 * sh + 2)

                @pl.when(ev)
                def _add_all():
                    def aj(j, _):
                        sl = pl.ds(j * LANE, LANE)
                        for s in rows:
                            pv = pbs[r][s, 0, sl]
                            for b in range(NSEQ):
                                obs[r][s, b, sl] = obs[r][s, b, sl] + pv
                        return 0

                    lax.fori_loop(0, VPR, aj, 0)

                @pl.when(jnp.logical_not(ev))
                def _add_odd():
                    def aj(j, _):
                        sl = pl.ds(j * LANE, LANE)
                        for s in rows:
                            pv = pbs[r][s, 0, sl]
                            for b in range(1, NSEQ, 2):
                                obs[r][s, b, sl] = obs[r][s, b, sl] + pv
                        return 0

                    lax.fori_loop(0, VPR, aj, 0)

                pltpu.make_async_copy(
                    obs[r].at[pl.ds(2 * sh, 2)],
                    out_hbm.at[pl.ds(ss + 2 * sh, 2)], out_sems[r]).start()

        return 0

    lax.fori_loop(0, NSLAB // NBUF, group, 0)

    for r in range(NBUF):
        out_copy(r, 0).wait()


_pe_pad_kernel = functools.partial(
    pl.kernel,
    out_type=jax.ShapeDtypeStruct((MAXLEN, NSEQ, EMB), jnp.float32),
    mesh=plsc.VectorSubcoreMesh(core_axis_name="c", subcore_axis_name="s",
                                num_cores=2, num_subcores=16),
    scratch_types=[
        pltpu.VMEM((SROWS, NSEQ, EMB), jnp.float32),
        pltpu.VMEM((SROWS, NSEQ, EMB), jnp.float32),
        pltpu.VMEM((SROWS, NSEQ, EMB), jnp.float32),
        pltpu.VMEM((SROWS, 1, EMB), jnp.float32),
        pltpu.VMEM((SROWS, 1, EMB), jnp.float32),
        pltpu.VMEM((SROWS, 1, EMB), jnp.float32),
        pltpu.SemaphoreType.DMA, pltpu.SemaphoreType.DMA, pltpu.SemaphoreType.DMA,
        pltpu.SemaphoreType.DMA, pltpu.SemaphoreType.DMA, pltpu.SemaphoreType.DMA,
    ],
)(_pe_pad_body)


def kernel(x, length, pe):
    del length
    x3 = x.reshape(TOTAL, 1, EMB)
    return _pe_pad_kernel(x3, pe)


# speedup vs baseline: 1.0040x; 1.0040x over previous
import functools

import jax
import jax.numpy as jnp
from jax import lax
from jax.experimental import pallas as pl
from jax.experimental.pallas import tpu as pltpu
from jax.experimental.pallas import tpu_sc as plsc

EMB = 512
NSEQ = 16
MAXLEN = 3072
TOTAL = 32768
LEN_EVEN = 1024
PAIR = 4096

NW = 32
SROWS = 4
NSLAB_LO = LEN_EVEN // NW // SROWS
NSLAB = NSLAB_LO + (MAXLEN - LEN_EVEN) // NW // SROWS
NBUF = 3
LANE = 16
VPR = EMB // LANE

_X_OFF = [(b >> 1) * PAIR + (b & 1) * LEN_EVEN for b in range(NSEQ)]


def _pe_pad_body(x_hbm, pe_hbm, out_hbm,
                 ob0, ob1, ob2, pb0, pb1, pb2,
                 si0, si1, si2, so0, so1, so2):
    wid = lax.axis_index("s") * 2 + lax.axis_index("c")

    obs = (ob0, ob1, ob2)
    pbs = (pb0, pb1, pb2)
    in_sems = (si0, si1, si2)
    out_sems = (so0, so1, so2)

    zero = jnp.zeros((LANE,), jnp.float32)
    lo0 = wid * (LEN_EVEN // NW)
    hi0 = LEN_EVEN + wid * ((MAXLEN - LEN_EVEN) // NW)

    def slab_ss(k):
        return jnp.where(k < NSLAB_LO,
                         lo0 + k * SROWS,
                         hi0 + (k - NSLAB_LO) * SROWS)

    def out_copy(par, ss):
        return pltpu.make_async_copy(
            obs[par], out_hbm.at[pl.ds(ss, SROWS)], out_sems[par])

    def fire_reads(par, k):
        ss = slab_ss(k)
        pltpu.make_async_copy(
            pe_hbm.at[pl.ds(ss, SROWS)], pbs[par], in_sems[par]).start()
        for b in range(1, NSEQ, 2):
            pltpu.make_async_copy(
                x_hbm.at[pl.ds(_X_OFF[b] + ss, SROWS)],
                obs[par].at[:, pl.ds(b, 1)], in_sems[par]).start()

        @pl.when(k < NSLAB_LO)
        def _():
            for b in range(0, NSEQ, 2):
                pltpu.make_async_copy(
                    x_hbm.at[pl.ds(_X_OFF[b] + ss, SROWS)],
                    obs[par].at[:, pl.ds(b, 1)], in_sems[par]).start()

    fire_reads(0, 0)

    def group(g, _):
        for r in range(NBUF):
            k = NBUF * g + r
            ss = slab_ss(k)
            ev = k < NSLAB_LO

            r1 = (r + 1) % NBUF

            @pl.when(k + 1 < NSLAB)
            def _():
                @pl.when(k + 1 >= NBUF)
                def _():
                    out_copy(r1, 0).wait()

                fire_reads(r1, k + 1)

            @pl.when((k >= NSLAB_LO) & (k < NSLAB_LO + NBUF))
            def _zero_even():
                def zj(j, _):
                    sl = pl.ds(j * LANE, LANE)
                    for s in range(SROWS):
                        for b in range(0, NSEQ, 2):
                            obs[r][s, b, sl] = zero
                    return 0

                lax.fori_loop(0, VPR, zj, 0)

            pltpu.make_async_copy(
                pe_hbm.at[pl.ds(0, SROWS)], pbs[r], in_sems[r]).wait()

            @pl.when(ev)
            def _():
                pltpu.make_async_copy(
                    out_hbm.at[pl.ds(0, SROWS)], obs[r], in_sems[r]).wait()

            @pl.when(jnp.logical_not(ev))
            def _():
                pltpu.make_async_copy(
                    out_hbm.at[pl.ds(0, SROWS), pl.ds(0, NSEQ // 2)],
                    obs[r].at[:, pl.ds(0, NSEQ // 2)], in_sems[r]).wait()

            @pl.when(ev)
            def _add_all():
                def aj(j, _):
                    sl = pl.ds(j * LANE, LANE)
                    for s in range(SROWS):
                        pv = pbs[r][s, 0, sl]
                        for b in range(NSEQ):
                            obs[r][s, b, sl] = obs[r][s, b, sl] + pv
                    return 0

                lax.fori_loop(0, VPR, aj, 0)

            @pl.when(jnp.logical_not(ev))
            def _add_odd():
                def aj(j, _):
                    sl = pl.ds(j * LANE, LANE)
                    for s in range(SROWS):
                        pv = pbs[r][s, 0, sl]
                        for b in range(1, NSEQ, 2):
                            obs[r][s, b, sl] = obs[r][s, b, sl] + pv
                    return 0

                lax.fori_loop(0, VPR, aj, 0)

            out_copy(r, ss).start()

        return 0

    lax.fori_loop(0, NSLAB // NBUF, group, 0)

    for r in range(NBUF):
        out_copy(r, 0).wait()


_pe_pad_kernel = functools.partial(
    pl.kernel,
    out_type=jax.ShapeDtypeStruct((MAXLEN, NSEQ, EMB), jnp.float32),
    mesh=plsc.VectorSubcoreMesh(core_axis_name="c", subcore_axis_name="s",
                                num_cores=2, num_subcores=16),
    scratch_types=[
        pltpu.VMEM((SROWS, NSEQ, EMB), jnp.float32),
        pltpu.VMEM((SROWS, NSEQ, EMB), jnp.float32),
        pltpu.VMEM((SROWS, NSEQ, EMB), jnp.float32),
        pltpu.VMEM((SROWS, 1, EMB), jnp.float32),
        pltpu.VMEM((SROWS, 1, EMB), jnp.float32),
        pltpu.VMEM((SROWS, 1, EMB), jnp.float32),
        pltpu.SemaphoreType.DMA, pltpu.SemaphoreType.DMA, pltpu.SemaphoreType.DMA,
        pltpu.SemaphoreType.DMA, pltpu.SemaphoreType.DMA, pltpu.SemaphoreType.DMA,
    ],
)(_pe_pad_body)


def kernel(x, length, pe):
    del length
    x3 = x.reshape(TOTAL, 1, EMB)
    return _pe_pad_kernel(x3, pe)
